# Initial kernel scaffold; baseline (speedup 1.0000x reference)
import jax
import jax.numpy as jnp
from jax.experimental import pallas as pl


def kernel(x, weights, bias, connections):
    raise NotImplementedError("write your pallas kernel here")


# speedup vs baseline: 6.0897x; 6.0897x over previous
import functools

import jax
import jax.numpy as jnp
from jax import lax
from jax.experimental import pallas as pl
from jax.experimental.pallas import tpu as pltpu
from jax.experimental.pallas import tpu_sc as plsc

LANES = 16
NC = 2
NS = 16
NW = NC * NS
CHUNK = 8192


def _sc_body(nchunks, n_in, n_out, bpw,
             rows_h, cols_h, w_h, x_h, bias_h, out_h,
             x_v, acc_v, rows_b0, cols_b0, w_b0, rows_b1, cols_b1, w_b1,
             sem_x, sem_a, sem_b):
  cid = lax.axis_index("c")
  sid = lax.axis_index("s")
  wid = sid * NC + cid

  xbase = wid * (bpw * n_in)
  cp_x = pltpu.async_copy(x_h.at[pl.ds(xbase, bpw * n_in)], x_v, sem_x)

  sems = (sem_a, sem_b)
  bufs = ((rows_b0, cols_b0, w_b0), (rows_b1, cols_b1, w_b1))
  pending = [
      pltpu.async_copy(rows_h.at[pl.ds(0, CHUNK)], rows_b0, sem_a),
      pltpu.async_copy(cols_h.at[pl.ds(0, CHUNK)], cols_b0, sem_a),
      pltpu.async_copy(w_h.at[pl.ds(0, CHUNK)], w_b0, sem_a),
  ]

  for b in range(bpw):
    pltpu.sync_copy(bias_h, acc_v.at[pl.ds(b * n_out, n_out)])
  cp_x.wait()

  for g in range(nchunks):
    slot = g % 2
    for cp in pending:
      cp.wait()
    if g + 1 < nchunks:
      nxt = slot ^ 1
      off = (g + 1) * CHUNK
      sem = sems[nxt]
      pending = [
          pltpu.async_copy(rows_h.at[pl.ds(off, CHUNK)], bufs[nxt][0], sem),
          pltpu.async_copy(cols_h.at[pl.ds(off, CHUNK)], bufs[nxt][1], sem),
          pltpu.async_copy(w_h.at[pl.ds(off, CHUNK)], bufs[nxt][2], sem),
      ]
    else:
      pending = []

    rb, cb, wb = bufs[slot]

    def inner(i, carry):
      o = pl.multiple_of(i * LANES, LANES)
      rv = rb[pl.ds(o, LANES)]
      cv = cb[pl.ds(o, LANES)]
      wv = wb[pl.ds(o, LANES)]
      for b in range(bpw):
        xv = plsc.load_gather(x_v, [rv + b * n_in])
        plsc.addupdate_scatter(acc_v, [cv + b * n_out], wv * xv)
      return carry

    lax.fori_loop(0, CHUNK // LANES, inner, 0)

  pltpu.sync_copy(acc_v, out_h.at[pl.ds(wid * (bpw * n_out), bpw * n_out)])


def kernel(x, weights, bias, connections):
  batch, n_in = x.shape
  n_out = bias.shape[0]
  nnz = weights.shape[0]
  bpw = batch // NW

  nchunks = -(-nnz // CHUNK)
  pad = nchunks * CHUNK - nnz

  rows = connections[:, 0]
  cols = connections[:, 1]
  if pad:
    zi = jnp.zeros((pad,), jnp.int32)
    rows = jnp.concatenate([rows, zi])
    cols = jnp.concatenate([cols, zi])
    weights = jnp.concatenate([weights, jnp.zeros((pad,), jnp.float32)])

  mesh = plsc.VectorSubcoreMesh(
      core_axis_name="c", subcore_axis_name="s", num_cores=NC,
      num_subcores=NS)
  body = functools.partial(_sc_body, nchunks, n_in, n_out, bpw)
  out_flat = pl.kernel(
      body,
      out_type=jax.ShapeDtypeStruct((batch * n_out,), jnp.float32),
      mesh=mesh,
      compiler_params=pltpu.CompilerParams(needs_layout_passes=False),
      scratch_types=[
          pltpu.VMEM((bpw * n_in,), jnp.float32),
          pltpu.VMEM((bpw * n_out,), jnp.float32),
          pltpu.VMEM((CHUNK,), jnp.int32),
          pltpu.VMEM((CHUNK,), jnp.int32),
          pltpu.VMEM((CHUNK,), jnp.float32),
          pltpu.VMEM((CHUNK,), jnp.int32),
          pltpu.VMEM((CHUNK,), jnp.int32),
          pltpu.VMEM((CHUNK,), jnp.float32),
          pltpu.SemaphoreType.DMA,
          pltpu.SemaphoreType.DMA,
          pltpu.SemaphoreType.DMA,
      ],
  )(rows, cols, weights, x.reshape(-1), bias.reshape(-1))
  return out_flat.reshape(batch, n_out)


# speedup vs baseline: 13.9620x; 2.2927x over previous
import functools

import jax
import jax.numpy as jnp
from jax import lax
from jax.experimental import pallas as pl
from jax.experimental.pallas import tpu as pltpu
from jax.experimental.pallas import tpu_sc as plsc

LANES = 16
NC = 2
NS = 16
NW = NC * NS
CHUNK = 8192
UNROLL = 8


def _sc_body(nchunks, n_in, n_out, bpw,
             rc_h, w_h, x_h, bias_h, out_h,
             x_v, acc_v, rc_b0, w_b0, rc_b1, w_b1,
             sem_x, sem_a, sem_b):
  cid = lax.axis_index("c")
  sid = lax.axis_index("s")
  wid = sid * NC + cid

  cp_x = [
      pltpu.async_copy(
          x_h.at[pl.ds((wid * bpw + b) * n_in, n_in)], x_v[b], sem_x)
      for b in range(bpw)
  ]

  sems = (sem_a, sem_b)
  bufs = ((rc_b0, w_b0), (rc_b1, w_b1))
  pending = [
      pltpu.async_copy(rc_h.at[pl.ds(0, CHUNK)], rc_b0, sem_a),
      pltpu.async_copy(w_h.at[pl.ds(0, CHUNK)], w_b0, sem_a),
  ]

  for b in range(bpw):
    pltpu.sync_copy(bias_h, acc_v[b])
  for cp in cp_x:
    cp.wait()

  for g in range(nchunks):
    slot = g % 2
    for cp in pending:
      cp.wait()
    if g + 1 < nchunks:
      nxt = slot ^ 1
      off = (g + 1) * CHUNK
      sem = sems[nxt]
      pending = [
          pltpu.async_copy(rc_h.at[pl.ds(off, CHUNK)], bufs[nxt][0], sem),
          pltpu.async_copy(w_h.at[pl.ds(off, CHUNK)], bufs[nxt][1], sem),
      ]
    else:
      pending = []

    rcb, wb = bufs[slot]

    @plsc.parallel_loop(0, CHUNK // LANES, unroll=UNROLL)
    def _(i):
      o = pl.multiple_of(i * LANES, LANES)
      rcv = rcb[pl.ds(o, LANES)]
      wv = wb[pl.ds(o, LANES)]
      rv = lax.bitwise_and(rcv, jnp.int32(0xFFFF))
      cv = lax.shift_right_logical(rcv, jnp.int32(16))
      for b in range(bpw):
        xv = plsc.load_gather(x_v[b], [rv])
        plsc.addupdate_scatter(acc_v[b], [cv], wv * xv)

  for b in range(bpw):
    pltpu.sync_copy(acc_v[b], out_h.at[pl.ds((wid * bpw + b) * n_out, n_out)])


def kernel(x, weights, bias, connections):
  batch, n_in = x.shape
  n_out = bias.shape[0]
  nnz = weights.shape[0]
  bpw = batch // NW

  nchunks = -(-nnz // CHUNK)
  pad = nchunks * CHUNK - nnz

  rc = lax.shift_left(connections[:, 1], 16) | connections[:, 0]
  if pad:
    rc = jnp.concatenate([rc, jnp.zeros((pad,), jnp.int32)])
    weights = jnp.concatenate([weights, jnp.zeros((pad,), jnp.float32)])

  mesh = plsc.VectorSubcoreMesh(
      core_axis_name="c", subcore_axis_name="s", num_cores=NC,
      num_subcores=NS)
  body = functools.partial(_sc_body, nchunks, n_in, n_out, bpw)
  out_flat = pl.kernel(
      body,
      out_type=jax.ShapeDtypeStruct((batch * n_out,), jnp.float32),
      mesh=mesh,
      compiler_params=pltpu.CompilerParams(needs_layout_passes=False),
      scratch_types=[
          [pltpu.VMEM((n_in,), jnp.float32) for _ in range(bpw)],
          [pltpu.VMEM((n_out,), jnp.float32) for _ in range(bpw)],
          pltpu.VMEM((CHUNK,), jnp.int32),
          pltpu.VMEM((CHUNK,), jnp.float32),
          pltpu.VMEM((CHUNK,), jnp.int32),
          pltpu.VMEM((CHUNK,), jnp.float32),
          pltpu.SemaphoreType.DMA,
          pltpu.SemaphoreType.DMA,
          pltpu.SemaphoreType.DMA,
      ],
  )(rc, weights, x.reshape(-1), bias.reshape(-1))
  return out_flat.reshape(batch, n_out)
